# Initial kernel scaffold; baseline (speedup 1.0000x reference)
import jax
import jax.numpy as jnp
from jax.experimental import pallas as pl


def kernel(pred, labels):
    raise NotImplementedError("write your pallas kernel here")


# speedup vs baseline: 1949.1919x; 1949.1919x over previous
import numpy as np
import jax
import jax.numpy as jnp
from jax.experimental import pallas as pl

_SMOOTH = 0.001
_KAPPA = float(np.sqrt(3.0) / 8.0)


def _point_areas(Xp0, Xp1):
    A = None
    for Xp in (Xp0, Xp1):
        H = jnp.abs(Xp[:, :-1] - Xp[:, 1:])
        V = jnp.abs(Xp[:-1, :] - Xp[1:, :])
        t = H[:-1, :] + H[1:, :] + V[:, :-1] + V[:, 1:]
        A = t if A is None else A + t
    Z = jnp.abs(Xp0 - Xp1)
    A = A + Z[:-1, :-1] + Z[:-1, 1:] + Z[1:, :-1] + Z[1:, 1:]
    return _KAPPA * A


def _dice_body(pred_ref, lab_ref, out_ref):
    zs = pred_ref.shape[0]
    S = jax.nn.sigmoid(pred_ref[...])
    S = jnp.pad(S, ((0, 0), (1, 1), (1, 1)))
    L = jnp.pad(lab_ref[...], ((0, 0), (1, 1), (1, 1)))
    num = jnp.float32(0.0)
    den = jnp.float32(0.0)
    for z in range(zs - 1):
        pa = _point_areas(S[z], S[z + 1])
        la = _point_areas(L[z], L[z + 1])
        both = jnp.logical_and(pa > 0, la > 0)
        num = num + jnp.sum(jnp.where(both, pa + la, 0.0))
        den = den + jnp.sum(pa) + jnp.sum(la)
    dice = 1.0 - (num + _SMOOTH) / (den + _SMOOTH)
    out_ref[...] = jnp.full((1, 1), dice, jnp.float32)


def kernel(pred, labels):
    B = pred.shape[0]
    dices = []
    for b in range(B):
        out = pl.pallas_call(
            _dice_body,
            out_shape=jax.ShapeDtypeStruct((1, 1), jnp.float32),
        )(pred[b], labels[b].astype(jnp.float32))
        dices.append(out[0, 0])
    return jnp.mean(jnp.stack(dices))
